# baseline (speedup 1.0000x reference)
import functools

import jax
import jax.numpy as jnp
from jax import lax
from jax.experimental import pallas as pl
from jax.experimental.pallas import tpu as pltpu
from jax.experimental.pallas import tpu_sc as plsc

L = 16


@functools.lru_cache(maxsize=None)
def _build(B, S, D, MAXS):
    info = plsc.get_sparse_core_info()
    NC, NS = info.num_cores, info.num_subcores
    NW = NC * NS
    assert S % NW == 0 and D % L == 0 and (D & (D - 1)) == 0
    Dlog = D.bit_length() - 1
    rows_w = S // NW
    P = 16
    while rows_w % P:
        P //= 2
    n_chunks = rows_w // P
    CW = P * D
    STEPS = n_chunks * B

    mesh = plsc.VectorSubcoreMesh(core_axis_name="c", subcore_axis_name="s")

    NB = 5
    PD = NB - 2

    @functools.partial(
        pl.kernel,
        out_type=jax.ShapeDtypeStruct((B, S, D), jnp.float32),
        mesh=mesh,
        scratch_types=(
            [pltpu.VMEM((P, D), jnp.float32)] * (NB + 2)
            + [pltpu.SemaphoreType.DMA] * (2 * NB + 2)
        ),
    )
    def k(emb_hbm, pos_hbm, out_hbm, *bufs):
        ebuf = list(bufs[:NB])
        pbuf = list(bufs[NB:NB + 2])
        lsem = list(bufs[NB + 2:2 * NB + 2])
        ssem = list(bufs[2 * NB + 2:3 * NB + 2])
        qsem = list(bufs[3 * NB + 2:3 * NB + 4])

        wid = lax.axis_index("s") * NC + lax.axis_index("c")
        s_base = wid * rows_w

        def row0(i):
            return s_base + (i // B) * P

        def start_load(i):
            return pltpu.async_copy(
                emb_hbm.at[i % B, pl.ds(row0(i), P), :],
                ebuf[i % NB], lsem[i % NB])

        ld = [None] * NB
        st = [None] * NB
        pw = [None] * 2

        pw[0] = pltpu.async_copy(
            pos_hbm.at[pl.ds(row0(0), P), :], pbuf[0], qsem[0])
        for j in range(min(PD, STEPS)):
            ld[j % NB] = start_load(j)

        for i in range(STEPS):
            cs, b = divmod(i, B)
            bi = i % NB
            nbi = (i + PD) % NB
            if b == 0:
                if cs + 1 < n_chunks:
                    pw[(cs + 1) % 2] = pltpu.async_copy(
                        pos_hbm.at[pl.ds(s_base + (cs + 1) * P, P), :],
                        pbuf[(cs + 1) % 2], qsem[(cs + 1) % 2])
                pw[cs % 2].wait()
            if i + PD < STEPS:
                if st[nbi] is not None:
                    st[nbi].wait()
                ld[nbi] = start_load(i + PD)
            ld[bi].wait()
            eb = ebuf[bi]
            pb = pbuf[cs % 2]

            @plsc.parallel_loop(0, CW, step=L, unroll=2)
            def body(o):
                r = lax.shift_right_logical(o, Dlog)
                c = pl.multiple_of(lax.bitwise_and(o, D - 1), L)
                plsc.addupdate(eb.at[r, pl.ds(c, L)], pb[r, pl.ds(c, L)])

            st[bi] = pltpu.async_copy(
                eb, out_hbm.at[b, pl.ds(row0(i), P), :], ssem[bi])

        for j in range(STEPS - min(NB, STEPS), STEPS):
            st[j % NB].wait()

    return k


def kernel(embeddings, pos_table):
    B, S, D = embeddings.shape
    MAXS = pos_table.shape[0]
    return _build(B, S, D, MAXS)(embeddings, pos_table)


# speedup vs baseline: 1.1952x; 1.1952x over previous
import functools

import jax
import jax.numpy as jnp
from jax import lax
from jax.experimental import pallas as pl
from jax.experimental.pallas import tpu as pltpu
from jax.experimental.pallas import tpu_sc as plsc

L = 16


@functools.lru_cache(maxsize=None)
def _build(B, S, D, MAXS):
    info = plsc.get_sparse_core_info()
    NC, NS = info.num_cores, info.num_subcores
    NW = NC * NS
    assert S % NW == 0 and D % L == 0 and (D & (D - 1)) == 0
    Dlog = D.bit_length() - 1
    rows_w = S // NW
    P = 16
    while rows_w % P:
        P //= 2
    n_chunks = rows_w // P
    CW = P * D
    STEPS = n_chunks * B

    mesh = plsc.VectorSubcoreMesh(core_axis_name="c", subcore_axis_name="s")

    NB = 5
    PD = NB - 2

    @functools.partial(
        pl.kernel,
        out_type=jax.ShapeDtypeStruct((B, S, D), jnp.float32),
        mesh=mesh,
        scratch_types=(
            [pltpu.VMEM((P, D), jnp.float32)] * (NB + 2)
            + [pltpu.SemaphoreType.DMA] * (2 * NB + 2)
        ),
    )
    def k(emb_hbm, pos_hbm, out_hbm, *bufs):
        ebuf = list(bufs[:NB])
        pbuf = list(bufs[NB:NB + 2])
        lsem = list(bufs[NB + 2:2 * NB + 2])
        ssem = list(bufs[2 * NB + 2:3 * NB + 2])
        qsem = list(bufs[3 * NB + 2:3 * NB + 4])

        wid = lax.axis_index("s") * NC + lax.axis_index("c")
        s_base = wid * rows_w

        def row0(i):
            return s_base + (i // B) * P

        def start_load(i):
            return pltpu.async_copy(
                emb_hbm.at[i % B, pl.ds(row0(i), P), :],
                ebuf[i % NB], lsem[i % NB])

        ld = [None] * NB
        st = [None] * NB
        pw = [None] * 2

        pw[0] = pltpu.async_copy(
            pos_hbm.at[pl.ds(row0(0), P), :], pbuf[0], qsem[0])
        for j in range(min(PD, STEPS)):
            ld[j % NB] = start_load(j)

        for i in range(STEPS):
            cs, b = divmod(i, B)
            bi = i % NB
            nbi = (i + PD) % NB
            if b == 0:
                if cs + 1 < n_chunks:
                    pw[(cs + 1) % 2] = pltpu.async_copy(
                        pos_hbm.at[pl.ds(s_base + (cs + 1) * P, P), :],
                        pbuf[(cs + 1) % 2], qsem[(cs + 1) % 2])
                pw[cs % 2].wait()
            if i + PD < STEPS:
                if st[nbi] is not None:
                    st[nbi][0].wait()
                    st[nbi][1].wait()
                ld[nbi] = start_load(i + PD)
            ld[bi].wait()
            eb = ebuf[bi]
            pb = pbuf[cs % 2]
            H = P // 2

            @plsc.parallel_loop(0, CW // 2, step=L, unroll=4)
            def body0(o):
                r = lax.shift_right_logical(o, Dlog)
                c = pl.multiple_of(lax.bitwise_and(o, D - 1), L)
                plsc.addupdate(eb.at[r, pl.ds(c, L)], pb[r, pl.ds(c, L)])

            st0 = pltpu.async_copy(
                eb.at[pl.ds(0, H), :],
                out_hbm.at[b, pl.ds(row0(i), H), :], ssem[bi])

            @plsc.parallel_loop(CW // 2, CW, step=L, unroll=4)
            def body1(o):
                r = lax.shift_right_logical(o, Dlog)
                c = pl.multiple_of(lax.bitwise_and(o, D - 1), L)
                plsc.addupdate(eb.at[r, pl.ds(c, L)], pb[r, pl.ds(c, L)])

            st1 = pltpu.async_copy(
                eb.at[pl.ds(H, H), :],
                out_hbm.at[b, pl.ds(row0(i) + H, H), :], ssem[bi])
            st[bi] = (st0, st1)

        for j in range(STEPS - min(NB, STEPS), STEPS):
            st[j % NB][0].wait()
            st[j % NB][1].wait()

    return k


def kernel(embeddings, pos_table):
    B, S, D = embeddings.shape
    MAXS = pos_table.shape[0]
    return _build(B, S, D, MAXS)(embeddings, pos_table)


# speedup vs baseline: 1.2118x; 1.0139x over previous
import functools

import jax
import jax.numpy as jnp
from jax import lax
from jax.experimental import pallas as pl
from jax.experimental.pallas import tpu as pltpu
from jax.experimental.pallas import tpu_sc as plsc

L = 16


@functools.lru_cache(maxsize=None)
def _build(B, S, D, MAXS):
    info = plsc.get_sparse_core_info()
    NC, NS = info.num_cores, info.num_subcores
    NW = NC * NS
    assert S % NW == 0 and D % L == 0 and (D & (D - 1)) == 0
    Dlog = D.bit_length() - 1
    rows_w = S // NW
    P = 16
    while rows_w % P:
        P //= 2
    n_chunks = rows_w // P
    CW = P * D
    STEPS = n_chunks * B

    mesh = plsc.VectorSubcoreMesh(core_axis_name="c", subcore_axis_name="s")

    NB = 4
    PD = 2

    @functools.partial(
        pl.kernel,
        out_type=jax.ShapeDtypeStruct((B, S, D), jnp.float32),
        mesh=mesh,
        scratch_types=(
            [pltpu.VMEM((P, D), jnp.float32)] * (NB + 2)
            + [pltpu.SemaphoreType.DMA] * (2 * NB + 2)
        ),
    )
    def k(emb_hbm, pos_hbm, out_hbm, *bufs):
        ebuf = list(bufs[:NB])
        pbuf = list(bufs[NB:NB + 2])
        lsem = list(bufs[NB + 2:2 * NB + 2])
        ssem = list(bufs[2 * NB + 2:3 * NB + 2])
        qsem = list(bufs[3 * NB + 2:3 * NB + 4])

        wid = lax.axis_index("s") * NC + lax.axis_index("c")
        s_base = wid * rows_w

        def row0(i):
            return s_base + (i // B) * P

        def start_load(i):
            return pltpu.async_copy(
                emb_hbm.at[i % B, pl.ds(row0(i), P), :],
                ebuf[i % NB], lsem[i % NB])

        ld = [None] * NB
        st = [None] * NB
        pw = [None] * 2

        pw[0] = pltpu.async_copy(
            pos_hbm.at[pl.ds(row0(0), P), :], pbuf[0], qsem[0])
        for j in range(min(PD, STEPS)):
            ld[j % NB] = start_load(j)

        for i in range(STEPS):
            cs, b = divmod(i, B)
            bi = i % NB
            nbi = (i + PD) % NB
            if b == 0:
                if cs + 1 < n_chunks:
                    pw[(cs + 1) % 2] = pltpu.async_copy(
                        pos_hbm.at[pl.ds(s_base + (cs + 1) * P, P), :],
                        pbuf[(cs + 1) % 2], qsem[(cs + 1) % 2])
                pw[cs % 2].wait()
            if i + PD < STEPS:
                if st[nbi] is not None:
                    st[nbi].wait()
                ld[nbi] = start_load(i + PD)
            ld[bi].wait()
            eb = ebuf[bi]
            pb = pbuf[cs % 2]

            @plsc.parallel_loop(0, CW, step=L, unroll=4)
            def body(o):
                r = lax.shift_right_logical(o, Dlog)
                c = pl.multiple_of(lax.bitwise_and(o, D - 1), L)
                plsc.addupdate(eb.at[r, pl.ds(c, L)], pb[r, pl.ds(c, L)])

            st[bi] = pltpu.async_copy(
                eb, out_hbm.at[b, pl.ds(row0(i), P), :], ssem[bi])

        for j in range(STEPS - min(NB, STEPS), STEPS):
            st[j % NB].wait()

    return k


def kernel(embeddings, pos_table):
    B, S, D = embeddings.shape
    MAXS = pos_table.shape[0]
    return _build(B, S, D, MAXS)(embeddings, pos_table)


# speedup vs baseline: 1.2304x; 1.0153x over previous
import functools

import jax
import jax.numpy as jnp
from jax import lax
from jax.experimental import pallas as pl
from jax.experimental.pallas import tpu as pltpu
from jax.experimental.pallas import tpu_sc as plsc

L = 16


@functools.lru_cache(maxsize=None)
def _build(B, S, D, MAXS):
    info = plsc.get_sparse_core_info()
    NC, NS = info.num_cores, info.num_subcores
    NW = NC * NS
    assert S % NW == 0 and D % L == 0 and (D & (D - 1)) == 0
    Dlog = D.bit_length() - 1
    rows_w = S // NW
    P = 16
    while rows_w % P:
        P //= 2
    n_chunks = rows_w // P
    CW = P * D
    STEPS = n_chunks * B

    mesh = plsc.VectorSubcoreMesh(core_axis_name="c", subcore_axis_name="s")

    NB = 5
    PD = NB - 2

    @functools.partial(
        pl.kernel,
        out_type=jax.ShapeDtypeStruct((B, S, D), jnp.float32),
        mesh=mesh,
        scratch_types=(
            [pltpu.VMEM((P, D), jnp.float32)] * (NB + 2)
            + [pltpu.SemaphoreType.DMA] * (2 * NB + 2)
        ),
    )
    def k(emb_hbm, pos_hbm, out_hbm, *bufs):
        ebuf = list(bufs[:NB])
        pbuf = list(bufs[NB:NB + 2])
        lsem = list(bufs[NB + 2:2 * NB + 2])
        ssem = list(bufs[2 * NB + 2:3 * NB + 2])
        qsem = list(bufs[3 * NB + 2:3 * NB + 4])

        wid = lax.axis_index("s") * NC + lax.axis_index("c")
        s_base = wid * rows_w

        def row0(i):
            return s_base + (i // B) * P

        def start_load(i):
            return pltpu.async_copy(
                emb_hbm.at[i % B, pl.ds(row0(i), P), :],
                ebuf[i % NB], lsem[i % NB])

        ld = [None] * NB
        st = [None] * NB
        pw = [None] * 2

        pw[0] = pltpu.async_copy(
            pos_hbm.at[pl.ds(row0(0), P), :], pbuf[0], qsem[0])
        for j in range(min(PD, STEPS)):
            ld[j % NB] = start_load(j)

        for i in range(STEPS):
            cs, b = divmod(i, B)
            bi = i % NB
            nbi = (i + PD) % NB
            if b == 0:
                if cs + 1 < n_chunks:
                    pw[(cs + 1) % 2] = pltpu.async_copy(
                        pos_hbm.at[pl.ds(s_base + (cs + 1) * P, P), :],
                        pbuf[(cs + 1) % 2], qsem[(cs + 1) % 2])
                pw[cs % 2].wait()
            if i + PD < STEPS:
                if st[nbi] is not None:
                    st[nbi].wait()
                ld[nbi] = start_load(i + PD)
            ld[bi].wait()
            eb = ebuf[bi]
            pb = pbuf[cs % 2]

            @plsc.parallel_loop(0, CW, step=L, unroll=4)
            def body(o):
                r = lax.shift_right_logical(o, Dlog)
                c = pl.multiple_of(lax.bitwise_and(o, D - 1), L)
                plsc.addupdate(eb.at[r, pl.ds(c, L)], pb[r, pl.ds(c, L)])

            st[bi] = pltpu.async_copy(
                eb, out_hbm.at[b, pl.ds(row0(i), P), :], ssem[bi])

        for j in range(STEPS - min(NB, STEPS), STEPS):
            st[j % NB].wait()

    return k


def kernel(embeddings, pos_table):
    B, S, D = embeddings.shape
    MAXS = pos_table.shape[0]
    return _build(B, S, D, MAXS)(embeddings, pos_table)
